# Initial kernel scaffold; baseline (speedup 1.0000x reference)
import jax
import jax.numpy as jnp
from jax.experimental import pallas as pl


def kernel(objs, obj_vecs, pred_vecs, boxes, s_idx, o_idx, params):
    raise NotImplementedError("write your pallas kernel here")


# speedup vs baseline: 2.9924x; 2.9924x over previous
import functools

import jax
import jax.numpy as jnp
from jax import lax
from jax.experimental import pallas as pl
from jax.experimental.pallas import tpu as pltpu
from jax.experimental.pallas import tpu_sc as plsc

NC = 2
NT = 16
CH = 128

_f32 = jnp.float32


def _lrelu(x):
    return jnp.where(x >= 0, x, 0.2 * x)


def _mesh():
    return plsc.VectorSubcoreMesh(core_axis_name="c", subcore_axis_name="s",
                                  num_cores=NC, num_subcores=NT)


def _sc_gather(ov, s_idx, o_idx):
    N, D = ov.shape
    E = s_idx.shape[0]
    nchunk = E // CH
    iters = -(-nchunk // (NC * NT))

    def body(ov_hbm, s_hbm, o_hbm, outs_hbm, outo_hbm,
             idx1, rows1, idx2, rows2, sem1, sem2):
        wid = lax.axis_index("s") * NC + lax.axis_index("c")

        def step(t, carry):
            c = wid + t * (NC * NT)

            @pl.when(c < nchunk)
            def _do():
                base = c * CH
                pltpu.sync_copy(s_hbm.at[pl.ds(base, CH)], idx1)
                pltpu.sync_copy(o_hbm.at[pl.ds(base, CH)], idx2)
                cp1 = pltpu.async_copy(ov_hbm.at[idx1], rows1, sem1)
                cp2 = pltpu.async_copy(ov_hbm.at[idx2], rows2, sem2)
                cp1.wait()
                cp2.wait()
                pltpu.sync_copy(rows1, outs_hbm.at[pl.ds(base, CH)])
                pltpu.sync_copy(rows2, outo_hbm.at[pl.ds(base, CH)])

            return carry

        lax.fori_loop(0, iters, step, 0)

    fn = pl.kernel(
        body,
        out_type=(jax.ShapeDtypeStruct((E, D), _f32),
                  jax.ShapeDtypeStruct((E, D), _f32)),
        mesh=_mesh(),
        scratch_types=[
            pltpu.VMEM((CH,), jnp.int32), pltpu.VMEM((CH, D), _f32),
            pltpu.VMEM((CH,), jnp.int32), pltpu.VMEM((CH, D), _f32),
            pltpu.SemaphoreType.DMA, pltpu.SemaphoreType.DMA,
        ],
    )
    return fn(ov, s_idx, o_idx)


def _sc_scatter(ns, no, s_idx, o_idx, zeros128):
    E, H = ns.shape
    N = zeros128.shape[0]
    nchunk = E // CH
    iters = -(-nchunk // NT)
    RB = 400
    nrchunk = N // RB
    riters = -(-nrchunk // NT)

    def row_sweep(sid, fn):
        def rstep(t, carry):
            r = sid + t * NT

            @pl.when(r < nrchunk)
            def _do():
                fn(r * RB)

            return carry

        lax.fori_loop(0, riters, rstep, 0)

    def body(ns_hbm, no_hbm, s_hbm, o_hbm, z_hbm, pooled_hbm,
             idx1, v1, idx2, v2, acc):
        cid = lax.axis_index("c")
        sid = lax.axis_index("s")
        for p in range(2):
            col0 = p * (2 * 128) + cid * 128
            row_sweep(sid, lambda r0: pltpu.sync_copy(
                z_hbm.at[pl.ds(r0, RB)], acc.at[pl.ds(r0, RB)]))
            plsc.subcore_barrier()

            def step(t, carry):
                c = sid + t * NT

                @pl.when(c < nchunk)
                def _do():
                    base = c * CH
                    pltpu.sync_copy(s_hbm.at[pl.ds(base, CH)], idx1)
                    pltpu.sync_copy(ns_hbm.at[pl.ds(base, CH), pl.ds(col0, 128)], v1)
                    pltpu.sync_copy(v1, acc.at[idx1], add=True)
                    pltpu.sync_copy(o_hbm.at[pl.ds(base, CH)], idx2)
                    pltpu.sync_copy(no_hbm.at[pl.ds(base, CH), pl.ds(col0, 128)], v2)
                    pltpu.sync_copy(v2, acc.at[idx2], add=True)

                return carry

            lax.fori_loop(0, iters, step, 0)
            plsc.subcore_barrier()
            row_sweep(sid, lambda r0: pltpu.sync_copy(
                acc.at[pl.ds(r0, RB)],
                pooled_hbm.at[pl.ds(r0, RB), pl.ds(col0, 128)]))
            plsc.subcore_barrier()

    fn = pl.kernel(
        body,
        out_type=jax.ShapeDtypeStruct((N, H), _f32),
        mesh=_mesh(),
        scratch_types=[
            pltpu.VMEM((CH,), jnp.int32), pltpu.VMEM((CH, 128), _f32),
            pltpu.VMEM((CH,), jnp.int32), pltpu.VMEM((CH, 128), _f32),
            pltpu.VMEM_SHARED((N, 128), _f32),
        ],
    )
    return fn(ns, no, s_idx, o_idx, zeros128)


def _sc_counts(s_idx, o_idx, ones_hbm, zeros128):
    E = s_idx.shape[0]
    N = zeros128.shape[0]
    nchunk = E // CH
    iters = -(-nchunk // NT)
    RB = 400
    nrchunk = N // RB
    riters = -(-nrchunk // NT)

    def body(s_hbm, o_hbm, on_hbm, z_hbm, cnt_hbm, idx1, ones_v, acc):
        cid = lax.axis_index("c")
        sid = lax.axis_index("s")

        def row_sweep(fn):
            def rstep(t, carry):
                r = sid + t * NT

                @pl.when(r < nrchunk)
                def _do():
                    fn(r * RB)

                return carry

            lax.fori_loop(0, riters, rstep, 0)

        @pl.when(cid == 0)
        def _sc0():
            pltpu.sync_copy(on_hbm, ones_v)
            row_sweep(lambda r0: pltpu.sync_copy(
                z_hbm.at[pl.ds(r0, RB)], acc.at[pl.ds(r0, RB)]))
            plsc.subcore_barrier()

            def step(t, carry):
                c = sid + t * NT

                @pl.when(c < nchunk)
                def _do():
                    base = c * CH
                    pltpu.sync_copy(s_hbm.at[pl.ds(base, CH)], idx1)
                    pltpu.sync_copy(ones_v, acc.at[idx1], add=True)
                    pltpu.sync_copy(o_hbm.at[pl.ds(base, CH)], idx1)
                    pltpu.sync_copy(ones_v, acc.at[idx1], add=True)

                return carry

            lax.fori_loop(0, iters, step, 0)
            plsc.subcore_barrier()
            row_sweep(lambda r0: pltpu.sync_copy(
                acc.at[pl.ds(r0, RB)], cnt_hbm.at[pl.ds(r0, RB)]))

    fn = pl.kernel(
        body,
        out_type=jax.ShapeDtypeStruct((N, 128), _f32),
        mesh=_mesh(),
        scratch_types=[
            pltpu.VMEM((CH,), jnp.int32), pltpu.VMEM((CH, 128), _f32),
            pltpu.VMEM_SHARED((N, 128), _f32),
        ],
    )
    return fn(s_idx, o_idx, ones_hbm, zeros128)


def _edge_mlp(ovs, pv, ovo, w1a, w1b, w1c, b1, w2s, b2s, w2p, b2p, w2o, b2o):
    E = ovs.shape[0]
    BE = 1000
    H = w1a.shape[1]
    Dout = w2p.shape[1]

    def body(ovs_ref, pv_ref, ovo_ref, w1a_ref, w1b_ref, w1c_ref, b1_ref,
             w2s_ref, b2s_ref, w2p_ref, b2p_ref, w2o_ref, b2o_ref,
             ns_ref, np_ref, no_ref):
        h = _lrelu(jnp.dot(ovs_ref[...], w1a_ref[...], preferred_element_type=_f32)
                   + jnp.dot(pv_ref[...], w1b_ref[...], preferred_element_type=_f32)
                   + jnp.dot(ovo_ref[...], w1c_ref[...], preferred_element_type=_f32)
                   + b1_ref[...])
        ns_ref[...] = _lrelu(jnp.dot(h, w2s_ref[...], preferred_element_type=_f32) + b2s_ref[...])
        np_ref[...] = _lrelu(jnp.dot(h, w2p_ref[...], preferred_element_type=_f32) + b2p_ref[...])
        no_ref[...] = _lrelu(jnp.dot(h, w2o_ref[...], preferred_element_type=_f32) + b2o_ref[...])

    def full(a):
        return pl.BlockSpec(a.shape, lambda i: (0,) * a.ndim)

    row = lambda d: pl.BlockSpec((BE, d), lambda i: (i, 0))
    return pl.pallas_call(
        body,
        grid=(E // BE,),
        in_specs=[row(ovs.shape[1]), row(pv.shape[1]), row(ovo.shape[1]),
                  full(w1a), full(w1b), full(w1c), full(b1),
                  full(w2s), full(b2s), full(w2p), full(b2p), full(w2o), full(b2o)],
        out_specs=[row(H), row(Dout), row(H)],
        out_shape=[jax.ShapeDtypeStruct((E, H), _f32),
                   jax.ShapeDtypeStruct((E, Dout), _f32),
                   jax.ShapeDtypeStruct((E, H), _f32)],
    )(ovs, pv, ovo, w1a, w1b, w1c, b1, w2s, b2s, w2p, b2p, w2o, b2o)


def _node_mlp(pooled, counts16, wa, ba, wb, bb):
    N, H = pooled.shape
    BN = 1000
    Dout = wb.shape[1]

    def body(p_ref, c_ref, wa_ref, ba_ref, wb_ref, bb_ref, out_ref):
        cnt = jnp.maximum(c_ref[...][:, :1], 1.0)
        x = p_ref[...] / cnt
        h = _lrelu(jnp.dot(x, wa_ref[...], preferred_element_type=_f32) + ba_ref[...])
        out_ref[...] = _lrelu(jnp.dot(h, wb_ref[...], preferred_element_type=_f32) + bb_ref[...])

    def full(a):
        return pl.BlockSpec(a.shape, lambda i: (0,) * a.ndim)

    return pl.pallas_call(
        body,
        grid=(N // BN,),
        in_specs=[pl.BlockSpec((BN, H), lambda i: (i, 0)),
                  pl.BlockSpec((BN, 128), lambda i: (i, 0)),
                  full(wa), full(ba), full(wb), full(bb)],
        out_specs=pl.BlockSpec((BN, Dout), lambda i: (i, 0)),
        out_shape=jax.ShapeDtypeStruct((N, Dout), _f32),
    )(pooled, counts16, wa, ba, wb, bb)


def _decode_pre(ov, boxes, wg1a, wg1b, bg1, wg2, bg2, wbb, bbb):
    N = ov.shape[0]

    def body(ov_ref, bx_ref, wg1a_ref, wg1b_ref, bg1_ref, wg2_ref, bg2_ref,
             wbb_ref, bbb_ref, a2_ref, b2_ref, hb_ref):
        pre = jnp.dot(ov_ref[...], wg1a_ref[...], preferred_element_type=_f32) + bg1_ref[...]
        bxw = jnp.dot(bx_ref[...], wg1b_ref[...], preferred_element_type=_f32)
        a = _lrelu(pre)
        b = _lrelu(pre + bxw)
        a2_ref[...] = _lrelu(jnp.dot(a, wg2_ref[...], preferred_element_type=_f32) + bg2_ref[...])
        b2_ref[...] = _lrelu(jnp.dot(b, wg2_ref[...], preferred_element_type=_f32) + bg2_ref[...])
        hb_ref[...] = _lrelu(jnp.dot(bx_ref[...], wbb_ref[...], preferred_element_type=_f32)
                             + bbb_ref[...])

    def full(a):
        return pl.BlockSpec(a.shape, lambda: (0,) * a.ndim)

    args = (ov, boxes, wg1a, wg1b, bg1, wg2, bg2, wbb, bbb)
    return pl.pallas_call(
        body,
        in_specs=[full(a) for a in args],
        out_specs=[pl.BlockSpec((N, 128), lambda: (0, 0))] * 3,
        out_shape=[jax.ShapeDtypeStruct((N, 128), _f32)] * 3,
    )(*args)


def _decode_steps(a2r, b2r, hbr, eps, wc, bc, wh1, wh2, bh, wmu, bmu, wvar, bvar,
                  wd1a, wd1b, bd1, wd2, bd2, wd3, bd3,
                  wp1, bp1, wp2, bp2, wp3, bp3, wpmu, bpmu, wpvar, bpvar):
    S = a2r.shape[1]
    M = a2r.shape[0]

    def body(a2_ref, b2_ref, hb_ref, eps_ref, wc_ref, bc_ref, wh1_ref, wh2_ref,
             bh_ref, wmu_ref, bmu_ref, wvar_ref, bvar_ref, wd1a_ref, wd1b_ref,
             bd1_ref, wd2_ref, bd2_ref, wd3_ref, bd3_ref, wp1_ref, bp1_ref,
             wp2_ref, bp2_ref, wp3_ref, bp3_ref, wpmu_ref, bpmu_ref,
             wpvar_ref, bpvar_ref, out_ref):
        for k in range(S):
            acc = jnp.zeros((M, 128), _f32)
            for j in range(S):
                acc = acc + (b2_ref[:, j, :] if j < k else a2_ref[:, j, :])
            c = acc * (1.0 / S)
            hb = hb_ref[:, k, :]
            hc = _lrelu(jnp.dot(c, wc_ref[...], preferred_element_type=_f32) + bc_ref[...])
            hh = _lrelu(jnp.dot(hb, wh1_ref[...], preferred_element_type=_f32)
                        + jnp.dot(hc, wh2_ref[...], preferred_element_type=_f32) + bh_ref[...])
            mu = _lrelu(jnp.dot(hh, wmu_ref[...], preferred_element_type=_f32) + bmu_ref[...])
            var = _lrelu(jnp.dot(hh, wvar_ref[...], preferred_element_type=_f32) + bvar_ref[...])
            z = eps_ref[k, :, :] * jnp.exp(var * 0.5) + mu
            d1 = _lrelu(jnp.dot(z, wd1a_ref[...], preferred_element_type=_f32)
                        + jnp.dot(c, wd1b_ref[...], preferred_element_type=_f32) + bd1_ref[...])
            d2 = _lrelu(jnp.dot(d1, wd2_ref[...], preferred_element_type=_f32) + bd2_ref[...])
            pred = jnp.dot(d2, wd3_ref[...], preferred_element_type=_f32) + bd3_ref[...]
            hp = _lrelu(jnp.dot(c, wp1_ref[...], preferred_element_type=_f32) + bp1_ref[...])
            hp = _lrelu(jnp.dot(hp, wp2_ref[...], preferred_element_type=_f32) + bp2_ref[...])
            hp = _lrelu(jnp.dot(hp, wp3_ref[...], preferred_element_type=_f32) + bp3_ref[...])
            mup = _lrelu(jnp.dot(hp, wpmu_ref[...], preferred_element_type=_f32) + bpmu_ref[...])
            varp = _lrelu(jnp.dot(hp, wpvar_ref[...], preferred_element_type=_f32) + bpvar_ref[...])
            res = jnp.concatenate([pred, mu, var, mup, varp], axis=1)
            out_ref[k, :, :] = res

    def full(a):
        return pl.BlockSpec(a.shape, lambda: (0,) * a.ndim)

    args = (a2r, b2r, hbr, eps, wc, bc, wh1, wh2, bh, wmu, bmu, wvar, bvar,
            wd1a, wd1b, bd1, wd2, bd2, wd3, bd3,
            wp1, bp1, wp2, bp2, wp3, bp3, wpmu, bpmu, wpvar, bpvar)
    return pl.pallas_call(
        body,
        in_specs=[full(a) for a in args],
        out_specs=pl.BlockSpec((S, M, 132), lambda: (0, 0, 0)),
        out_shape=jax.ShapeDtypeStruct((S, M, 132), _f32),
    )(*args)


def kernel(objs, obj_vecs, pred_vecs, boxes, s_idx, o_idx, params):
    N = obj_vecs.shape[0]
    E = s_idx.shape[0]
    s_idx = s_idx.astype(jnp.int32)
    o_idx = o_idx.astype(jnp.int32)
    zeros128 = jnp.zeros((N, 128), _f32)
    ones128 = jnp.ones((CH, 128), _f32)

    counts16 = _sc_counts(s_idx, o_idx, ones128, zeros128)

    ov, pv = obj_vecs, pred_vecs
    for layer in params['gconv']:
        W1, b1 = layer['net1'][0]
        W2, b2 = layer['net1'][1]
        Din = ov.shape[1]
        H = layer['net2'][0][0].shape[0]
        Dout = layer['net2'][1][0].shape[1]
        pad = 128 - Din
        ov_g = ov if pad == 0 else jnp.pad(ov, ((0, 0), (0, pad)))
        w1a = jnp.pad(W1[:Din], ((0, pad), (0, 0)))
        w1c = jnp.pad(W1[2 * Din:], ((0, pad), (0, 0)))
        ovs, ovo = _sc_gather(ov_g, s_idx, o_idx)
        ns, np_, no = _edge_mlp(
            ovs, pv, ovo,
            w1a, W1[Din:2 * Din], w1c, b1[None, :],
            W2[:, :H], b2[None, :H],
            W2[:, H:H + Dout], b2[None, H:H + Dout],
            W2[:, H + Dout:], b2[None, H + Dout:])
        pooled = _sc_scatter(ns, no, s_idx, o_idx, zeros128)
        (wa, ba), (wb, bb) = layer['net2']
        ov = _node_mlp(pooled, counts16, wa, ba[None, :], wb, bb[None, :])
        pv = np_

    S = 10
    num_scenes = objs.shape[0] // S
    (Wg1, bg1), (Wg2, bg2) = params['g_update']
    Wbb, bbb = params['enc_bb']
    a2, b2_, hb = _decode_pre(ov, boxes, Wg1[:128], Wg1[128:], bg1[None, :],
                              Wg2, bg2[None, :], Wbb, bbb[None, :])
    a2r = a2.reshape(num_scenes, S, 128)
    b2r = b2_.reshape(num_scenes, S, 128)
    hbr = hb.reshape(num_scenes, S, 128)
    ekey = jax.random.key(42)
    eps = jnp.stack([jax.random.normal(jax.random.fold_in(ekey, k),
                                       (num_scenes, 32), _f32) for k in range(S)])

    Wc, bc = params['enc_c']
    Wh, bh = params['enc_h']
    Wmu, bmu = params['enc_mu']
    Wvar, bvar = params['enc_var']
    (Wd1, bd1), (Wd2, bd2), (Wd3, bd3) = params['dec']
    (Wp1, bp1), (Wp2, bp2), (Wp3, bp3) = params['prior']
    Wpmu, bpmu = params['prior_mu']
    Wpvar, bpvar = params['prior_var']
    out = _decode_steps(
        a2r, b2r, hbr, eps,
        Wc, bc[None, :], Wh[:128], Wh[128:], bh[None, :],
        Wmu, bmu[None, :], Wvar, bvar[None, :],
        Wd1[:32], Wd1[32:], bd1[None, :], Wd2, bd2[None, :], Wd3, bd3[None, :],
        Wp1, bp1[None, :], Wp2, bp2[None, :], Wp3, bp3[None, :],
        Wpmu, bpmu[None, :], Wpvar, bpvar[None, :])
    return out + (jnp.sum(objs) * 0).astype(_f32)


# speedup vs baseline: 4.1931x; 1.4012x over previous
import functools

import jax
import jax.numpy as jnp
from jax import lax
from jax.experimental import pallas as pl
from jax.experimental.pallas import tpu as pltpu
from jax.experimental.pallas import tpu_sc as plsc

NC = 2
NT = 16
CH = 128

_f32 = jnp.float32


def _lrelu(x):
    return jnp.where(x >= 0, x, 0.2 * x)


def _mesh():
    return plsc.VectorSubcoreMesh(core_axis_name="c", subcore_axis_name="s",
                                  num_cores=NC, num_subcores=NT)


def _sc_gather(ov, s_idx, o_idx):
    N, D = ov.shape
    E = s_idx.shape[0]
    nchunk = E // CH
    iters = -(-nchunk // (NC * NT))

    def body(ov_hbm, s_hbm, o_hbm, outs_hbm, outo_hbm,
             idx1, rows1, idx2, rows2, sem1, sem2):
        wid = lax.axis_index("s") * NC + lax.axis_index("c")

        def step(t, carry):
            c = wid + t * (NC * NT)

            @pl.when(c < nchunk)
            def _do():
                base = c * CH
                pltpu.sync_copy(s_hbm.at[pl.ds(base, CH)], idx1)
                pltpu.sync_copy(o_hbm.at[pl.ds(base, CH)], idx2)
                cp1 = pltpu.async_copy(ov_hbm.at[idx1], rows1, sem1)
                cp2 = pltpu.async_copy(ov_hbm.at[idx2], rows2, sem2)
                cp1.wait()
                cp2.wait()
                pltpu.sync_copy(rows1, outs_hbm.at[pl.ds(base, CH)])
                pltpu.sync_copy(rows2, outo_hbm.at[pl.ds(base, CH)])

            return carry

        lax.fori_loop(0, iters, step, 0)

    fn = pl.kernel(
        body,
        out_type=(jax.ShapeDtypeStruct((E, D), _f32),
                  jax.ShapeDtypeStruct((E, D), _f32)),
        mesh=_mesh(),
        scratch_types=[
            pltpu.VMEM((CH,), jnp.int32), pltpu.VMEM((CH, D), _f32),
            pltpu.VMEM((CH,), jnp.int32), pltpu.VMEM((CH, D), _f32),
            pltpu.SemaphoreType.DMA, pltpu.SemaphoreType.DMA,
        ],
    )
    return fn(ov, s_idx, o_idx)


def _sc_scatter(ns, no, s2, o2, zeros128):
    E, H = ns.shape
    N = zeros128.shape[0]
    SUP = CH
    SUPC = SUP // CH
    nsup = E // SUP
    NSUP = 2 * nsup
    iters = -(-NSUP // NT)
    RB = 400
    nrchunk = N // RB
    riters = -(-nrchunk // NT)

    def row_sweep(sid, fn):
        def rstep(t, carry):
            r = sid + t * NT

            @pl.when(r < nrchunk)
            def _do():
                fn(r * RB)

            return carry

        lax.fori_loop(0, riters, rstep, 0)

    def body(ns_hbm, no_hbm, s_hbm, o_hbm, z_hbm, pooled_hbm,
             idx0, idx1, v0, v1, acc, si0, si1, sv0, sv1, sa0, sa1):
        cid = lax.axis_index("c")
        sid = lax.axis_index("s")
        idxb = (idx0, idx1)
        vb = (v0, v1)
        sib = (si0, si1)
        svb = (sv0, sv1)
        sab = (sa0, sa1)

        def with_srcs(t, col0, fn):
            u = sid + t * NT

            @pl.when(u < nsup)
            def _s():
                fn(s_hbm.at[pl.ds(u * SUPC, SUPC)],
                   ns_hbm.at[pl.ds(u * SUP, SUP), pl.ds(col0, 128)])

            @pl.when(jnp.logical_and(u >= nsup, u < NSUP))
            def _o():
                uo = u - nsup
                fn(o_hbm.at[pl.ds(uo * SUPC, SUPC)],
                   no_hbm.at[pl.ds(uo * SUP, SUP), pl.ds(col0, 128)])

        def start_load(t, b, col0):
            with_srcs(t, col0, lambda isrc, vsrc: (
                pltpu.async_copy(isrc, idxb[b], sib[b]),
                pltpu.async_copy(vsrc, vb[b], svb[b])))

        def wait_load(t, b, col0):
            with_srcs(t, col0, lambda isrc, vsrc: (
                pltpu.make_async_copy(isrc, idxb[b], sib[b]).wait(),
                pltpu.make_async_copy(vsrc, vb[b], svb[b]).wait()))

        def adds(t, b, start):
            u = sid + t * NT

            @pl.when(u < NSUP)
            def _do():
                for j in range(SUPC):
                    src = vb[b].at[pl.ds(j * CH, CH)]
                    dst = acc.at[idxb[b].at[j]]
                    if start:
                        pltpu.async_copy(src, dst, sab[b], add=True)
                    else:
                        pltpu.make_async_copy(src, dst, sab[b]).wait()

        def step(k, carry):
            col0 = carry
            for b in range(2):
                t = 2 * k + b
                wait_load(t, b, col0)
                adds(t, b, True)
                adds(t, b, False)
                start_load(t + 2, b, col0)
            return carry

        for p in range(2):
            col0 = p * (2 * 128) + cid * 128
            row_sweep(sid, lambda r0: pltpu.sync_copy(
                z_hbm.at[pl.ds(r0, RB)], acc.at[pl.ds(r0, RB)]))
            plsc.subcore_barrier()
            start_load(0, 0, col0)
            start_load(1, 1, col0)
            lax.fori_loop(0, -(-iters // 2), step, col0)
            plsc.subcore_barrier()
            row_sweep(sid, lambda r0: pltpu.sync_copy(
                acc.at[pl.ds(r0, RB)],
                pooled_hbm.at[pl.ds(r0, RB), pl.ds(col0, 128)]))
            plsc.subcore_barrier()

    fn = pl.kernel(
        body,
        out_type=jax.ShapeDtypeStruct((N, H), _f32),
        mesh=_mesh(),
        scratch_types=[
            pltpu.VMEM((SUPC, CH), jnp.int32), pltpu.VMEM((SUPC, CH), jnp.int32),
            pltpu.VMEM((SUP, 128), _f32), pltpu.VMEM((SUP, 128), _f32),
            pltpu.VMEM_SHARED((N, 128), _f32),
            pltpu.SemaphoreType.DMA, pltpu.SemaphoreType.DMA,
            pltpu.SemaphoreType.DMA, pltpu.SemaphoreType.DMA,
            pltpu.SemaphoreType.DMA, pltpu.SemaphoreType.DMA,
        ],
    )
    return fn(ns, no, s2, o2, zeros128)


def _sc_counts(s_idx, o_idx, ones_hbm, zeros128):
    E = s_idx.shape[0]
    N = zeros128.shape[0]
    nchunk = E // CH
    iters = -(-nchunk // NT)
    RB = 400
    nrchunk = N // RB
    riters = -(-nrchunk // NT)

    def body(s_hbm, o_hbm, on_hbm, z_hbm, cnt_hbm, idx1, ones_v, acc):
        cid = lax.axis_index("c")
        sid = lax.axis_index("s")

        def row_sweep(fn):
            def rstep(t, carry):
                r = sid + t * NT

                @pl.when(r < nrchunk)
                def _do():
                    fn(r * RB)

                return carry

            lax.fori_loop(0, riters, rstep, 0)

        @pl.when(cid == 0)
        def _sc0():
            pltpu.sync_copy(on_hbm, ones_v)
            row_sweep(lambda r0: pltpu.sync_copy(
                z_hbm.at[pl.ds(r0, RB)], acc.at[pl.ds(r0, RB)]))
            plsc.subcore_barrier()

            def step(t, carry):
                c = sid + t * NT

                @pl.when(c < nchunk)
                def _do():
                    base = c * CH
                    pltpu.sync_copy(s_hbm.at[pl.ds(base, CH)], idx1)
                    pltpu.sync_copy(ones_v, acc.at[idx1], add=True)
                    pltpu.sync_copy(o_hbm.at[pl.ds(base, CH)], idx1)
                    pltpu.sync_copy(ones_v, acc.at[idx1], add=True)

                return carry

            lax.fori_loop(0, iters, step, 0)
            plsc.subcore_barrier()
            row_sweep(lambda r0: pltpu.sync_copy(
                acc.at[pl.ds(r0, RB)], cnt_hbm.at[pl.ds(r0, RB)]))

    fn = pl.kernel(
        body,
        out_type=jax.ShapeDtypeStruct((N, 128), _f32),
        mesh=_mesh(),
        scratch_types=[
            pltpu.VMEM((CH,), jnp.int32), pltpu.VMEM((CH, 128), _f32),
            pltpu.VMEM_SHARED((N, 128), _f32),
        ],
    )
    return fn(s_idx, o_idx, ones_hbm, zeros128)


def _edge_mlp(ovs, pv, ovo, w1a, w1b, w1c, b1, w2s, b2s, w2p, b2p, w2o, b2o):
    E = ovs.shape[0]
    BE = 1000
    H = w1a.shape[1]
    Dout = w2p.shape[1]

    def body(ovs_ref, pv_ref, ovo_ref, w1a_ref, w1b_ref, w1c_ref, b1_ref,
             w2s_ref, b2s_ref, w2p_ref, b2p_ref, w2o_ref, b2o_ref,
             ns_ref, np_ref, no_ref):
        h = _lrelu(jnp.dot(ovs_ref[...], w1a_ref[...], preferred_element_type=_f32)
                   + jnp.dot(pv_ref[...], w1b_ref[...], preferred_element_type=_f32)
                   + jnp.dot(ovo_ref[...], w1c_ref[...], preferred_element_type=_f32)
                   + b1_ref[...])
        ns_ref[...] = _lrelu(jnp.dot(h, w2s_ref[...], preferred_element_type=_f32) + b2s_ref[...])
        np_ref[...] = _lrelu(jnp.dot(h, w2p_ref[...], preferred_element_type=_f32) + b2p_ref[...])
        no_ref[...] = _lrelu(jnp.dot(h, w2o_ref[...], preferred_element_type=_f32) + b2o_ref[...])

    def full(a):
        return pl.BlockSpec(a.shape, lambda i: (0,) * a.ndim)

    row = lambda d: pl.BlockSpec((BE, d), lambda i: (i, 0))
    return pl.pallas_call(
        body,
        grid=(E // BE,),
        in_specs=[row(ovs.shape[1]), row(pv.shape[1]), row(ovo.shape[1]),
                  full(w1a), full(w1b), full(w1c), full(b1),
                  full(w2s), full(b2s), full(w2p), full(b2p), full(w2o), full(b2o)],
        out_specs=[row(H), row(Dout), row(H)],
        out_shape=[jax.ShapeDtypeStruct((E, H), _f32),
                   jax.ShapeDtypeStruct((E, Dout), _f32),
                   jax.ShapeDtypeStruct((E, H), _f32)],
    )(ovs, pv, ovo, w1a, w1b, w1c, b1, w2s, b2s, w2p, b2p, w2o, b2o)


def _node_mlp(pooled, counts16, wa, ba, wb, bb):
    N, H = pooled.shape
    BN = 1000
    Dout = wb.shape[1]

    def body(p_ref, c_ref, wa_ref, ba_ref, wb_ref, bb_ref, out_ref):
        cnt = jnp.maximum(c_ref[...][:, :1], 1.0)
        x = p_ref[...] / cnt
        h = _lrelu(jnp.dot(x, wa_ref[...], preferred_element_type=_f32) + ba_ref[...])
        out_ref[...] = _lrelu(jnp.dot(h, wb_ref[...], preferred_element_type=_f32) + bb_ref[...])

    def full(a):
        return pl.BlockSpec(a.shape, lambda i: (0,) * a.ndim)

    return pl.pallas_call(
        body,
        grid=(N // BN,),
        in_specs=[pl.BlockSpec((BN, H), lambda i: (i, 0)),
                  pl.BlockSpec((BN, 128), lambda i: (i, 0)),
                  full(wa), full(ba), full(wb), full(bb)],
        out_specs=pl.BlockSpec((BN, Dout), lambda i: (i, 0)),
        out_shape=jax.ShapeDtypeStruct((N, Dout), _f32),
    )(pooled, counts16, wa, ba, wb, bb)


def _decode_pre(ov, boxes, wg1a, wg1b, bg1, wg2, bg2, wbb, bbb):
    N = ov.shape[0]

    def body(ov_ref, bx_ref, wg1a_ref, wg1b_ref, bg1_ref, wg2_ref, bg2_ref,
             wbb_ref, bbb_ref, a2_ref, b2_ref, hb_ref):
        pre = jnp.dot(ov_ref[...], wg1a_ref[...], preferred_element_type=_f32) + bg1_ref[...]
        bxw = jnp.dot(bx_ref[...], wg1b_ref[...], preferred_element_type=_f32)
        a = _lrelu(pre)
        b = _lrelu(pre + bxw)
        a2_ref[...] = _lrelu(jnp.dot(a, wg2_ref[...], preferred_element_type=_f32) + bg2_ref[...])
        b2_ref[...] = _lrelu(jnp.dot(b, wg2_ref[...], preferred_element_type=_f32) + bg2_ref[...])
        hb_ref[...] = _lrelu(jnp.dot(bx_ref[...], wbb_ref[...], preferred_element_type=_f32)
                             + bbb_ref[...])

    def full(a):
        return pl.BlockSpec(a.shape, lambda: (0,) * a.ndim)

    args = (ov, boxes, wg1a, wg1b, bg1, wg2, bg2, wbb, bbb)
    return pl.pallas_call(
        body,
        in_specs=[full(a) for a in args],
        out_specs=[pl.BlockSpec((N, 128), lambda: (0, 0))] * 3,
        out_shape=[jax.ShapeDtypeStruct((N, 128), _f32)] * 3,
    )(*args)


def _decode_steps(a2r, b2r, hbr, eps, wc, bc, wh1, wh2, bh, wmu, bmu, wvar, bvar,
                  wd1a, wd1b, bd1, wd2, bd2, wd3, bd3,
                  wp1, bp1, wp2, bp2, wp3, bp3, wpmu, bpmu, wpvar, bpvar):
    S = a2r.shape[1]
    M = a2r.shape[0]

    def body(a2_ref, b2_ref, hb_ref, eps_ref, wc_ref, bc_ref, wh1_ref, wh2_ref,
             bh_ref, wmu_ref, bmu_ref, wvar_ref, bvar_ref, wd1a_ref, wd1b_ref,
             bd1_ref, wd2_ref, bd2_ref, wd3_ref, bd3_ref, wp1_ref, bp1_ref,
             wp2_ref, bp2_ref, wp3_ref, bp3_ref, wpmu_ref, bpmu_ref,
             wpvar_ref, bpvar_ref, out_ref):
        for k in range(S):
            acc = jnp.zeros((M, 128), _f32)
            for j in range(S):
                acc = acc + (b2_ref[:, j, :] if j < k else a2_ref[:, j, :])
            c = acc * (1.0 / S)
            hb = hb_ref[:, k, :]
            hc = _lrelu(jnp.dot(c, wc_ref[...], preferred_element_type=_f32) + bc_ref[...])
            hh = _lrelu(jnp.dot(hb, wh1_ref[...], preferred_element_type=_f32)
                        + jnp.dot(hc, wh2_ref[...], preferred_element_type=_f32) + bh_ref[...])
            mu = _lrelu(jnp.dot(hh, wmu_ref[...], preferred_element_type=_f32) + bmu_ref[...])
            var = _lrelu(jnp.dot(hh, wvar_ref[...], preferred_element_type=_f32) + bvar_ref[...])
            z = eps_ref[k, :, :] * jnp.exp(var * 0.5) + mu
            d1 = _lrelu(jnp.dot(z, wd1a_ref[...], preferred_element_type=_f32)
                        + jnp.dot(c, wd1b_ref[...], preferred_element_type=_f32) + bd1_ref[...])
            d2 = _lrelu(jnp.dot(d1, wd2_ref[...], preferred_element_type=_f32) + bd2_ref[...])
            pred = jnp.dot(d2, wd3_ref[...], preferred_element_type=_f32) + bd3_ref[...]
            hp = _lrelu(jnp.dot(c, wp1_ref[...], preferred_element_type=_f32) + bp1_ref[...])
            hp = _lrelu(jnp.dot(hp, wp2_ref[...], preferred_element_type=_f32) + bp2_ref[...])
            hp = _lrelu(jnp.dot(hp, wp3_ref[...], preferred_element_type=_f32) + bp3_ref[...])
            mup = _lrelu(jnp.dot(hp, wpmu_ref[...], preferred_element_type=_f32) + bpmu_ref[...])
            varp = _lrelu(jnp.dot(hp, wpvar_ref[...], preferred_element_type=_f32) + bpvar_ref[...])
            res = jnp.concatenate([pred, mu, var, mup, varp], axis=1)
            out_ref[k, :, :] = res

    def full(a):
        return pl.BlockSpec(a.shape, lambda: (0,) * a.ndim)

    args = (a2r, b2r, hbr, eps, wc, bc, wh1, wh2, bh, wmu, bmu, wvar, bvar,
            wd1a, wd1b, bd1, wd2, bd2, wd3, bd3,
            wp1, bp1, wp2, bp2, wp3, bp3, wpmu, bpmu, wpvar, bpvar)
    return pl.pallas_call(
        body,
        in_specs=[full(a) for a in args],
        out_specs=pl.BlockSpec((S, M, 132), lambda: (0, 0, 0)),
        out_shape=jax.ShapeDtypeStruct((S, M, 132), _f32),
    )(*args)


def kernel(objs, obj_vecs, pred_vecs, boxes, s_idx, o_idx, params):
    N = obj_vecs.shape[0]
    E = s_idx.shape[0]
    s_idx = s_idx.astype(jnp.int32)
    o_idx = o_idx.astype(jnp.int32)
    s2 = s_idx.reshape(-1, CH)
    o2 = o_idx.reshape(-1, CH)
    zeros128 = jnp.zeros((N, 128), _f32)
    ones128 = jnp.ones((CH, 128), _f32)

    counts16 = _sc_counts(s_idx, o_idx, ones128, zeros128)

    ov, pv = obj_vecs, pred_vecs
    for layer in params['gconv']:
        W1, b1 = layer['net1'][0]
        W2, b2 = layer['net1'][1]
        Din = ov.shape[1]
        H = layer['net2'][0][0].shape[0]
        Dout = layer['net2'][1][0].shape[1]
        pad = 128 - Din
        ov_g = ov if pad == 0 else jnp.pad(ov, ((0, 0), (0, pad)))
        w1a = jnp.pad(W1[:Din], ((0, pad), (0, 0)))
        w1c = jnp.pad(W1[2 * Din:], ((0, pad), (0, 0)))
        ovs, ovo = _sc_gather(ov_g, s_idx, o_idx)
        ns, np_, no = _edge_mlp(
            ovs, pv, ovo,
            w1a, W1[Din:2 * Din], w1c, b1[None, :],
            W2[:, :H], b2[None, :H],
            W2[:, H:H + Dout], b2[None, H:H + Dout],
            W2[:, H + Dout:], b2[None, H + Dout:])
        pooled = _sc_scatter(ns, no, s2, o2, zeros128)
        (wa, ba), (wb, bb) = layer['net2']
        ov = _node_mlp(pooled, counts16, wa, ba[None, :], wb, bb[None, :])
        pv = np_

    S = 10
    num_scenes = objs.shape[0] // S
    (Wg1, bg1), (Wg2, bg2) = params['g_update']
    Wbb, bbb = params['enc_bb']
    a2, b2_, hb = _decode_pre(ov, boxes, Wg1[:128], Wg1[128:], bg1[None, :],
                              Wg2, bg2[None, :], Wbb, bbb[None, :])
    a2r = a2.reshape(num_scenes, S, 128)
    b2r = b2_.reshape(num_scenes, S, 128)
    hbr = hb.reshape(num_scenes, S, 128)
    ekey = jax.random.key(42)
    eps = jnp.stack([jax.random.normal(jax.random.fold_in(ekey, k),
                                       (num_scenes, 32), _f32) for k in range(S)])

    Wc, bc = params['enc_c']
    Wh, bh = params['enc_h']
    Wmu, bmu = params['enc_mu']
    Wvar, bvar = params['enc_var']
    (Wd1, bd1), (Wd2, bd2), (Wd3, bd3) = params['dec']
    (Wp1, bp1), (Wp2, bp2), (Wp3, bp3) = params['prior']
    Wpmu, bpmu = params['prior_mu']
    Wpvar, bpvar = params['prior_var']
    out = _decode_steps(
        a2r, b2r, hbr, eps,
        Wc, bc[None, :], Wh[:128], Wh[128:], bh[None, :],
        Wmu, bmu[None, :], Wvar, bvar[None, :],
        Wd1[:32], Wd1[32:], bd1[None, :], Wd2, bd2[None, :], Wd3, bd3[None, :],
        Wp1, bp1[None, :], Wp2, bp2[None, :], Wp3, bp3[None, :],
        Wpmu, bpmu[None, :], Wpvar, bpvar[None, :])
    return out + (jnp.sum(objs) * 0).astype(_f32)


# speedup vs baseline: 4.4140x; 1.0527x over previous
import functools

import jax
import jax.numpy as jnp
from jax import lax
from jax.experimental import pallas as pl
from jax.experimental.pallas import tpu as pltpu
from jax.experimental.pallas import tpu_sc as plsc

NC = 2
NT = 16
CH = 128

_f32 = jnp.float32


def _lrelu(x):
    return jnp.where(x >= 0, x, 0.2 * x)


def _mesh():
    return plsc.VectorSubcoreMesh(core_axis_name="c", subcore_axis_name="s",
                                  num_cores=NC, num_subcores=NT)


def _sc_gather(ov, s2p, o2p, E):
    N, D = ov.shape
    nchunk_pad = s2p.shape[0]
    nreal = E // CH
    NW = NC * NT
    IPW = nchunk_pad // NW

    def body(ov_hbm, s_hbm, o_hbm, outs_hbm, outo_hbm,
             isa, ioa, rs0, rs1, ro0, ro1, sg0, sg1, sw0, sw1):
        wid = lax.axis_index("s") * NC + lax.axis_index("c")
        base = wid * IPW
        rsb = (rs0, rs1)
        rob = (ro0, ro1)
        sgb = (sg0, sg1)
        swb = (sw0, sw1)
        pltpu.sync_copy(s_hbm.at[pl.ds(base, IPW)], isa)
        pltpu.sync_copy(o_hbm.at[pl.ds(base, IPW)], ioa)

        def gathers(t, b, start):
            @pl.when(jnp.logical_and(t < IPW, base + t < nreal))
            def _do():
                for rows, idx_all in ((rsb[b], isa), (rob[b], ioa)):
                    cp = pltpu.make_async_copy(
                        ov_hbm.at[idx_all.at[t]], rows, sgb[b])
                    if start:
                        pltpu.async_copy(ov_hbm.at[idx_all.at[t]], rows, sgb[b])
                    else:
                        cp.wait()

        def wbs(t, b, start):
            @pl.when(jnp.logical_and(t >= 0,
                                     jnp.logical_and(t < IPW,
                                                     base + t < nreal)))
            def _do():
                u = base + t
                for rows, out in ((rsb[b], outs_hbm), (rob[b], outo_hbm)):
                    cp = pltpu.make_async_copy(
                        rows, out.at[pl.ds(u * CH, CH)], swb[b])
                    if start:
                        pltpu.async_copy(rows, out.at[pl.ds(u * CH, CH)], swb[b])
                    else:
                        cp.wait()

        gathers(0, 0, True)

        def step(k, carry):
            for b in range(2):
                t = 2 * k + b
                gathers(t, b, False)
                wbs(t, b, True)
                wbs(t - 1, 1 - b, False)
                gathers(t + 1, 1 - b, True)
            return carry

        lax.fori_loop(0, -(-IPW // 2), step, 0)
        wbs(2 * (-(-IPW // 2)) - 1, 1, False)

    fn = pl.kernel(
        body,
        out_type=(jax.ShapeDtypeStruct((E, D), _f32),
                  jax.ShapeDtypeStruct((E, D), _f32)),
        mesh=_mesh(),
        scratch_types=[
            pltpu.VMEM((IPW, CH), jnp.int32), pltpu.VMEM((IPW, CH), jnp.int32),
            pltpu.VMEM((CH, D), _f32), pltpu.VMEM((CH, D), _f32),
            pltpu.VMEM((CH, D), _f32), pltpu.VMEM((CH, D), _f32),
            pltpu.SemaphoreType.DMA, pltpu.SemaphoreType.DMA,
            pltpu.SemaphoreType.DMA, pltpu.SemaphoreType.DMA,
        ],
    )
    return fn(ov, s2p, o2p)


def _pad_rows(x, mult):
    r = x.shape[0] % mult
    return x if r == 0 else jnp.pad(x, ((0, mult - r), (0, 0)))


def _sc_scatter(ns, no, s2, o2, zeros128):
    E, H = ns.shape
    N = zeros128.shape[0]
    SUP = CH
    SUPC = SUP // CH
    nsup = E // SUP
    NSUP = 2 * nsup
    iters = -(-NSUP // NT)
    RB = 400
    nrchunk = N // RB
    riters = -(-nrchunk // NT)

    def row_sweep(sid, fn):
        def rstep(t, carry):
            r = sid + t * NT

            @pl.when(r < nrchunk)
            def _do():
                fn(r * RB)

            return carry

        lax.fori_loop(0, riters, rstep, 0)

    def body(ns_hbm, no_hbm, s_hbm, o_hbm, z_hbm, pooled_hbm,
             idx0, idx1, v0, v1, acc, si0, si1, sv0, sv1, sa0, sa1):
        cid = lax.axis_index("c")
        sid = lax.axis_index("s")
        idxb = (idx0, idx1)
        vb = (v0, v1)
        sib = (si0, si1)
        svb = (sv0, sv1)
        sab = (sa0, sa1)

        def with_srcs(t, col0, fn):
            u = sid + t * NT

            @pl.when(u < nsup)
            def _s():
                fn(s_hbm.at[pl.ds(u * SUPC, SUPC)],
                   ns_hbm.at[pl.ds(u * SUP, SUP), pl.ds(col0, 128)])

            @pl.when(jnp.logical_and(u >= nsup, u < NSUP))
            def _o():
                uo = u - nsup
                fn(o_hbm.at[pl.ds(uo * SUPC, SUPC)],
                   no_hbm.at[pl.ds(uo * SUP, SUP), pl.ds(col0, 128)])

        def start_load(t, b, col0):
            with_srcs(t, col0, lambda isrc, vsrc: (
                pltpu.async_copy(isrc, idxb[b], sib[b]),
                pltpu.async_copy(vsrc, vb[b], svb[b])))

        def wait_load(t, b, col0):
            with_srcs(t, col0, lambda isrc, vsrc: (
                pltpu.make_async_copy(isrc, idxb[b], sib[b]).wait(),
                pltpu.make_async_copy(vsrc, vb[b], svb[b]).wait()))

        def adds(t, b, start):
            u = sid + t * NT

            @pl.when(u < NSUP)
            def _do():
                for j in range(SUPC):
                    src = vb[b].at[pl.ds(j * CH, CH)]
                    dst = acc.at[idxb[b].at[j]]
                    if start:
                        pltpu.async_copy(src, dst, sab[b], add=True)
                    else:
                        pltpu.make_async_copy(src, dst, sab[b]).wait()

        def step(k, carry):
            col0 = carry
            for b in range(2):
                t = 2 * k + b
                wait_load(t, b, col0)
                adds(t, b, True)
                adds(t, b, False)
                start_load(t + 2, b, col0)
            return carry

        for p in range(2):
            col0 = p * (2 * 128) + cid * 128
            row_sweep(sid, lambda r0: pltpu.sync_copy(
                z_hbm.at[pl.ds(r0, RB)], acc.at[pl.ds(r0, RB)]))
            plsc.subcore_barrier()
            start_load(0, 0, col0)
            start_load(1, 1, col0)
            lax.fori_loop(0, -(-iters // 2), step, col0)
            plsc.subcore_barrier()
            row_sweep(sid, lambda r0: pltpu.sync_copy(
                acc.at[pl.ds(r0, RB)],
                pooled_hbm.at[pl.ds(r0, RB), pl.ds(col0, 128)]))
            plsc.subcore_barrier()

    fn = pl.kernel(
        body,
        out_type=jax.ShapeDtypeStruct((N, H), _f32),
        mesh=_mesh(),
        scratch_types=[
            pltpu.VMEM((SUPC, CH), jnp.int32), pltpu.VMEM((SUPC, CH), jnp.int32),
            pltpu.VMEM((SUP, 128), _f32), pltpu.VMEM((SUP, 128), _f32),
            pltpu.VMEM_SHARED((N, 128), _f32),
            pltpu.SemaphoreType.DMA, pltpu.SemaphoreType.DMA,
            pltpu.SemaphoreType.DMA, pltpu.SemaphoreType.DMA,
            pltpu.SemaphoreType.DMA, pltpu.SemaphoreType.DMA,
        ],
    )
    return fn(ns, no, s2, o2, zeros128)


def _sc_counts(s_idx, o_idx, ones_hbm, zeros128):
    E = s_idx.shape[0]
    N = zeros128.shape[0]
    nchunk = E // CH
    iters = -(-nchunk // NT)
    RB = 400
    nrchunk = N // RB
    riters = -(-nrchunk // NT)

    def body(s_hbm, o_hbm, on_hbm, z_hbm, cnt_hbm, idx1, ones_v, acc):
        cid = lax.axis_index("c")
        sid = lax.axis_index("s")

        def row_sweep(fn):
            def rstep(t, carry):
                r = sid + t * NT

                @pl.when(r < nrchunk)
                def _do():
                    fn(r * RB)

                return carry

            lax.fori_loop(0, riters, rstep, 0)

        @pl.when(cid == 0)
        def _sc0():
            pltpu.sync_copy(on_hbm, ones_v)
            row_sweep(lambda r0: pltpu.sync_copy(
                z_hbm.at[pl.ds(r0, RB)], acc.at[pl.ds(r0, RB)]))
            plsc.subcore_barrier()

            def step(t, carry):
                c = sid + t * NT

                @pl.when(c < nchunk)
                def _do():
                    base = c * CH
                    pltpu.sync_copy(s_hbm.at[pl.ds(base, CH)], idx1)
                    pltpu.sync_copy(ones_v, acc.at[idx1], add=True)
                    pltpu.sync_copy(o_hbm.at[pl.ds(base, CH)], idx1)
                    pltpu.sync_copy(ones_v, acc.at[idx1], add=True)

                return carry

            lax.fori_loop(0, iters, step, 0)
            plsc.subcore_barrier()
            row_sweep(lambda r0: pltpu.sync_copy(
                acc.at[pl.ds(r0, RB)], cnt_hbm.at[pl.ds(r0, RB)]))

    fn = pl.kernel(
        body,
        out_type=jax.ShapeDtypeStruct((N, 128), _f32),
        mesh=_mesh(),
        scratch_types=[
            pltpu.VMEM((CH,), jnp.int32), pltpu.VMEM((CH, 128), _f32),
            pltpu.VMEM_SHARED((N, 128), _f32),
        ],
    )
    return fn(s_idx, o_idx, ones_hbm, zeros128)


def _edge_mlp(ovs, pv, ovo, w1a, w1b, w1c, b1, w2s, b2s, w2p, b2p, w2o, b2o):
    E = ovs.shape[0]
    BE = 1000
    H = w1a.shape[1]
    Dout = w2p.shape[1]

    def body(ovs_ref, pv_ref, ovo_ref, w1a_ref, w1b_ref, w1c_ref, b1_ref,
             w2s_ref, b2s_ref, w2p_ref, b2p_ref, w2o_ref, b2o_ref,
             ns_ref, np_ref, no_ref):
        h = _lrelu(jnp.dot(ovs_ref[...], w1a_ref[...], preferred_element_type=_f32)
                   + jnp.dot(pv_ref[...], w1b_ref[...], preferred_element_type=_f32)
                   + jnp.dot(ovo_ref[...], w1c_ref[...], preferred_element_type=_f32)
                   + b1_ref[...])
        ns_ref[...] = _lrelu(jnp.dot(h, w2s_ref[...], preferred_element_type=_f32) + b2s_ref[...])
        np_ref[...] = _lrelu(jnp.dot(h, w2p_ref[...], preferred_element_type=_f32) + b2p_ref[...])
        no_ref[...] = _lrelu(jnp.dot(h, w2o_ref[...], preferred_element_type=_f32) + b2o_ref[...])

    def full(a):
        return pl.BlockSpec(a.shape, lambda i: (0,) * a.ndim)

    row = lambda d: pl.BlockSpec((BE, d), lambda i: (i, 0))
    return pl.pallas_call(
        body,
        grid=(E // BE,),
        in_specs=[row(ovs.shape[1]), row(pv.shape[1]), row(ovo.shape[1]),
                  full(w1a), full(w1b), full(w1c), full(b1),
                  full(w2s), full(b2s), full(w2p), full(b2p), full(w2o), full(b2o)],
        out_specs=[row(H), row(Dout), row(H)],
        out_shape=[jax.ShapeDtypeStruct((E, H), _f32),
                   jax.ShapeDtypeStruct((E, Dout), _f32),
                   jax.ShapeDtypeStruct((E, H), _f32)],
    )(ovs, pv, ovo, w1a, w1b, w1c, b1, w2s, b2s, w2p, b2p, w2o, b2o)


def _node_mlp(pooled, counts16, wa, ba, wb, bb):
    N, H = pooled.shape
    BN = 1000
    Dout = wb.shape[1]

    def body(p_ref, c_ref, wa_ref, ba_ref, wb_ref, bb_ref, out_ref):
        cnt = jnp.maximum(c_ref[...][:, :1], 1.0)
        x = p_ref[...] / cnt
        h = _lrelu(jnp.dot(x, wa_ref[...], preferred_element_type=_f32) + ba_ref[...])
        out_ref[...] = _lrelu(jnp.dot(h, wb_ref[...], preferred_element_type=_f32) + bb_ref[...])

    def full(a):
        return pl.BlockSpec(a.shape, lambda i: (0,) * a.ndim)

    return pl.pallas_call(
        body,
        grid=(N // BN,),
        in_specs=[pl.BlockSpec((BN, H), lambda i: (i, 0)),
                  pl.BlockSpec((BN, 128), lambda i: (i, 0)),
                  full(wa), full(ba), full(wb), full(bb)],
        out_specs=pl.BlockSpec((BN, Dout), lambda i: (i, 0)),
        out_shape=jax.ShapeDtypeStruct((N, Dout), _f32),
    )(pooled, counts16, wa, ba, wb, bb)


def _decode_pre(ov, boxes, wg1a, wg1b, bg1, wg2, bg2, wbb, bbb):
    N = ov.shape[0]

    def body(ov_ref, bx_ref, wg1a_ref, wg1b_ref, bg1_ref, wg2_ref, bg2_ref,
             wbb_ref, bbb_ref, a2_ref, b2_ref, hb_ref):
        pre = jnp.dot(ov_ref[...], wg1a_ref[...], preferred_element_type=_f32) + bg1_ref[...]
        bxw = jnp.dot(bx_ref[...], wg1b_ref[...], preferred_element_type=_f32)
        a = _lrelu(pre)
        b = _lrelu(pre + bxw)
        a2_ref[...] = _lrelu(jnp.dot(a, wg2_ref[...], preferred_element_type=_f32) + bg2_ref[...])
        b2_ref[...] = _lrelu(jnp.dot(b, wg2_ref[...], preferred_element_type=_f32) + bg2_ref[...])
        hb_ref[...] = _lrelu(jnp.dot(bx_ref[...], wbb_ref[...], preferred_element_type=_f32)
                             + bbb_ref[...])

    def full(a):
        return pl.BlockSpec(a.shape, lambda: (0,) * a.ndim)

    args = (ov, boxes, wg1a, wg1b, bg1, wg2, bg2, wbb, bbb)
    return pl.pallas_call(
        body,
        in_specs=[full(a) for a in args],
        out_specs=[pl.BlockSpec((N, 128), lambda: (0, 0))] * 3,
        out_shape=[jax.ShapeDtypeStruct((N, 128), _f32)] * 3,
    )(*args)


def _decode_steps(a2r, b2r, hbr, eps, wc, bc, wh1, wh2, bh, wmu, bmu, wvar, bvar,
                  wd1a, wd1b, bd1, wd2, bd2, wd3, bd3,
                  wp1, bp1, wp2, bp2, wp3, bp3, wpmu, bpmu, wpvar, bpvar):
    S = a2r.shape[1]
    M = a2r.shape[0]

    def body(a2_ref, b2_ref, hb_ref, eps_ref, wc_ref, bc_ref, wh1_ref, wh2_ref,
             bh_ref, wmu_ref, bmu_ref, wvar_ref, bvar_ref, wd1a_ref, wd1b_ref,
             bd1_ref, wd2_ref, bd2_ref, wd3_ref, bd3_ref, wp1_ref, bp1_ref,
             wp2_ref, bp2_ref, wp3_ref, bp3_ref, wpmu_ref, bpmu_ref,
             wpvar_ref, bpvar_ref, out_ref):
        for k in range(S):
            acc = jnp.zeros((M, 128), _f32)
            for j in range(S):
                acc = acc + (b2_ref[:, j, :] if j < k else a2_ref[:, j, :])
            c = acc * (1.0 / S)
            hb = hb_ref[:, k, :]
            hc = _lrelu(jnp.dot(c, wc_ref[...], preferred_element_type=_f32) + bc_ref[...])
            hh = _lrelu(jnp.dot(hb, wh1_ref[...], preferred_element_type=_f32)
                        + jnp.dot(hc, wh2_ref[...], preferred_element_type=_f32) + bh_ref[...])
            mu = _lrelu(jnp.dot(hh, wmu_ref[...], preferred_element_type=_f32) + bmu_ref[...])
            var = _lrelu(jnp.dot(hh, wvar_ref[...], preferred_element_type=_f32) + bvar_ref[...])
            z = eps_ref[k, :, :] * jnp.exp(var * 0.5) + mu
            d1 = _lrelu(jnp.dot(z, wd1a_ref[...], preferred_element_type=_f32)
                        + jnp.dot(c, wd1b_ref[...], preferred_element_type=_f32) + bd1_ref[...])
            d2 = _lrelu(jnp.dot(d1, wd2_ref[...], preferred_element_type=_f32) + bd2_ref[...])
            pred = jnp.dot(d2, wd3_ref[...], preferred_element_type=_f32) + bd3_ref[...]
            hp = _lrelu(jnp.dot(c, wp1_ref[...], preferred_element_type=_f32) + bp1_ref[...])
            hp = _lrelu(jnp.dot(hp, wp2_ref[...], preferred_element_type=_f32) + bp2_ref[...])
            hp = _lrelu(jnp.dot(hp, wp3_ref[...], preferred_element_type=_f32) + bp3_ref[...])
            mup = _lrelu(jnp.dot(hp, wpmu_ref[...], preferred_element_type=_f32) + bpmu_ref[...])
            varp = _lrelu(jnp.dot(hp, wpvar_ref[...], preferred_element_type=_f32) + bpvar_ref[...])
            res = jnp.concatenate([pred, mu, var, mup, varp], axis=1)
            out_ref[k, :, :] = res

    def full(a):
        return pl.BlockSpec(a.shape, lambda: (0,) * a.ndim)

    args = (a2r, b2r, hbr, eps, wc, bc, wh1, wh2, bh, wmu, bmu, wvar, bvar,
            wd1a, wd1b, bd1, wd2, bd2, wd3, bd3,
            wp1, bp1, wp2, bp2, wp3, bp3, wpmu, bpmu, wpvar, bpvar)
    return pl.pallas_call(
        body,
        in_specs=[full(a) for a in args],
        out_specs=pl.BlockSpec((S, M, 132), lambda: (0, 0, 0)),
        out_shape=jax.ShapeDtypeStruct((S, M, 132), _f32),
    )(*args)


def kernel(objs, obj_vecs, pred_vecs, boxes, s_idx, o_idx, params):
    N = obj_vecs.shape[0]
    E = s_idx.shape[0]
    s_idx = s_idx.astype(jnp.int32)
    o_idx = o_idx.astype(jnp.int32)
    s2 = s_idx.reshape(-1, CH)
    o2 = o_idx.reshape(-1, CH)
    s2p = _pad_rows(s2, NC * NT)
    o2p = _pad_rows(o2, NC * NT)
    zeros128 = jnp.zeros((N, 128), _f32)
    ones128 = jnp.ones((CH, 128), _f32)

    counts16 = _sc_counts(s_idx, o_idx, ones128, zeros128)

    ov, pv = obj_vecs, pred_vecs
    for layer in params['gconv']:
        W1, b1 = layer['net1'][0]
        W2, b2 = layer['net1'][1]
        Din = ov.shape[1]
        H = layer['net2'][0][0].shape[0]
        Dout = layer['net2'][1][0].shape[1]
        pad = 128 - Din
        ov_g = ov if pad == 0 else jnp.pad(ov, ((0, 0), (0, pad)))
        w1a = jnp.pad(W1[:Din], ((0, pad), (0, 0)))
        w1c = jnp.pad(W1[2 * Din:], ((0, pad), (0, 0)))
        ovs, ovo = _sc_gather(ov_g, s2p, o2p, E)
        ns, np_, no = _edge_mlp(
            ovs, pv, ovo,
            w1a, W1[Din:2 * Din], w1c, b1[None, :],
            W2[:, :H], b2[None, :H],
            W2[:, H:H + Dout], b2[None, H:H + Dout],
            W2[:, H + Dout:], b2[None, H + Dout:])
        pooled = _sc_scatter(ns, no, s2, o2, zeros128)
        (wa, ba), (wb, bb) = layer['net2']
        ov = _node_mlp(pooled, counts16, wa, ba[None, :], wb, bb[None, :])
        pv = np_

    S = 10
    num_scenes = objs.shape[0] // S
    (Wg1, bg1), (Wg2, bg2) = params['g_update']
    Wbb, bbb = params['enc_bb']
    a2, b2_, hb = _decode_pre(ov, boxes, Wg1[:128], Wg1[128:], bg1[None, :],
                              Wg2, bg2[None, :], Wbb, bbb[None, :])
    a2r = a2.reshape(num_scenes, S, 128)
    b2r = b2_.reshape(num_scenes, S, 128)
    hbr = hb.reshape(num_scenes, S, 128)
    ekey = jax.random.key(42)
    eps = jnp.stack([jax.random.normal(jax.random.fold_in(ekey, k),
                                       (num_scenes, 32), _f32) for k in range(S)])

    Wc, bc = params['enc_c']
    Wh, bh = params['enc_h']
    Wmu, bmu = params['enc_mu']
    Wvar, bvar = params['enc_var']
    (Wd1, bd1), (Wd2, bd2), (Wd3, bd3) = params['dec']
    (Wp1, bp1), (Wp2, bp2), (Wp3, bp3) = params['prior']
    Wpmu, bpmu = params['prior_mu']
    Wpvar, bpvar = params['prior_var']
    out = _decode_steps(
        a2r, b2r, hbr, eps,
        Wc, bc[None, :], Wh[:128], Wh[128:], bh[None, :],
        Wmu, bmu[None, :], Wvar, bvar[None, :],
        Wd1[:32], Wd1[32:], bd1[None, :], Wd2, bd2[None, :], Wd3, bd3[None, :],
        Wp1, bp1[None, :], Wp2, bp2[None, :], Wp3, bp3[None, :],
        Wpmu, bpmu[None, :], Wpvar, bpvar[None, :])
    return out + (jnp.sum(objs) * 0).astype(_f32)
